# baseline (speedup 1.0000x reference)
import functools

import jax
import jax.numpy as jnp
from jax.experimental import pallas as pl
import jax.experimental.pallas.tpu as pltpu

LANES = 128
NEG = -1e30


def _router_kernel(x_ref, w_ref, b_ref, slots_ref, out_ref, acc_ref, *, grid, inv_n):
    i = pl.program_id(0)
    logits = (
        jnp.dot(x_ref[...].astype(jnp.bfloat16), w_ref[...],
                preferred_element_type=jnp.float32)
        + b_ref[...]
    )
    m = jnp.max(logits, axis=-1, keepdims=True)
    e = jnp.exp(logits - m)
    p = e / jnp.sum(e, axis=-1, keepdims=True)
    s = jnp.sum(p, axis=0, keepdims=True)

    @pl.when(i == 0)
    def _init():
        acc_ref[...] = s

    @pl.when(i > 0)
    def _acc():
        acc_ref[...] += s

    @pl.when(i == grid - 1)
    def _finish():
        nslots = out_ref.shape[0]
        avg = acc_ref[0:1, 0:nslots] * inv_n
        out_ref[...] = slots_ref[...] * avg.T


def kernel(input_stream, ssm_slots, msm_slots, router_w, router_b,
           compress_w, compress_b):
    del compress_w, compress_b
    n, d = input_stream.shape
    nslots = router_w.shape[0]

    chunk = 2048
    grid = n // chunk

    w_pad = (jnp.zeros((d, LANES), jnp.float32).at[:, :nslots]
             .set(router_w.T).astype(jnp.bfloat16))
    b_pad = jnp.full((1, LANES), NEG, jnp.float32).at[0, :nslots].set(router_b)
    slots = jnp.concatenate([ssm_slots, msm_slots], axis=0)

    out = pl.pallas_call(
        functools.partial(_router_kernel, grid=grid, inv_n=1.0 / n),
        grid=(grid,),
        in_specs=[
            pl.BlockSpec((chunk, d), lambda i: (i, 0)),
            pl.BlockSpec((d, LANES), lambda i: (0, 0)),
            pl.BlockSpec((1, LANES), lambda i: (0, 0)),
            pl.BlockSpec((nslots, d), lambda i: (0, 0)),
        ],
        out_specs=pl.BlockSpec((nslots, d), lambda i: (0, 0)),
        out_shape=jax.ShapeDtypeStruct((nslots, d), jnp.float32),
        scratch_shapes=[pltpu.VMEM((1, LANES), jnp.float32)],
    )(input_stream, w_pad, b_pad, slots)
    return out


# speedup vs baseline: 1.2150x; 1.2150x over previous
import functools

import jax
import jax.numpy as jnp
from jax.experimental import pallas as pl
import jax.experimental.pallas.tpu as pltpu

SUB = 8
NEG = -1e30


def _router_kernel(x_ref, w_ref, b_ref, slots_ref, out_ref, acc_ref, *, grid, inv_n):
    i = pl.program_id(0)
    lt = jax.lax.dot_general(
        w_ref[...], x_ref[...].astype(jnp.bfloat16),
        (((1,), (1,)), ((), ())),
        preferred_element_type=jnp.float32,
    ) + b_ref[...]
    m = jnp.max(lt, axis=0, keepdims=True)
    e = jnp.exp(lt - m)
    s = jnp.sum(e, axis=0, keepdims=True)
    p = e / s
    part = jnp.sum(p, axis=1, keepdims=True)

    @pl.when(i == 0)
    def _init():
        acc_ref[...] = part

    @pl.when(i > 0)
    def _acc():
        acc_ref[...] += part

    @pl.when(i == grid - 1)
    def _finish():
        nslots = out_ref.shape[0]
        out_ref[...] = slots_ref[...] * (acc_ref[0:nslots, :] * inv_n)


def kernel(input_stream, ssm_slots, msm_slots, router_w, router_b,
           compress_w, compress_b):
    del compress_w, compress_b
    n, d = input_stream.shape
    nslots = router_w.shape[0]

    chunk = 2048
    grid = n // chunk

    w_pad = (jnp.zeros((SUB, d), jnp.float32).at[:nslots, :]
             .set(router_w).astype(jnp.bfloat16))
    b_pad = jnp.full((SUB, 1), NEG, jnp.float32).at[:nslots, 0].set(router_b)
    slots = jnp.concatenate([ssm_slots, msm_slots], axis=0)

    out = pl.pallas_call(
        functools.partial(_router_kernel, grid=grid, inv_n=1.0 / n),
        grid=(grid,),
        in_specs=[
            pl.BlockSpec((chunk, d), lambda i: (i, 0)),
            pl.BlockSpec((SUB, d), lambda i: (0, 0)),
            pl.BlockSpec((SUB, 1), lambda i: (0, 0)),
            pl.BlockSpec((nslots, d), lambda i: (0, 0)),
        ],
        out_specs=pl.BlockSpec((nslots, d), lambda i: (0, 0)),
        out_shape=jax.ShapeDtypeStruct((nslots, d), jnp.float32),
        scratch_shapes=[pltpu.VMEM((SUB, 1), jnp.float32)],
    )(input_stream, w_pad, b_pad, slots)
    return out


# speedup vs baseline: 1.9503x; 1.6051x over previous
import functools

import jax
import jax.numpy as jnp
from jax.experimental import pallas as pl
import jax.experimental.pallas.tpu as pltpu

SUB = 8
NEG = -1e30


def _router_kernel(x_ref, w_ref, b_ref, slots_ref, out_ref, acc_ref, *, grid, inv_n):
    i = pl.program_id(0)
    lt = jax.lax.dot_general(
        w_ref[...], x_ref[...].astype(jnp.bfloat16),
        (((1,), (1,)), ((), ())),
        preferred_element_type=jnp.float32,
    ) + b_ref[...]
    m = jnp.max(lt, axis=0, keepdims=True)
    e = jnp.exp(lt - m)
    s = jnp.sum(e, axis=0, keepdims=True)
    p = e / s
    part = jnp.sum(p, axis=1, keepdims=True)

    @pl.when(i == 0)
    def _init():
        acc_ref[...] = part

    @pl.when(i > 0)
    def _acc():
        acc_ref[...] += part

    @pl.when(i == grid - 1)
    def _finish():
        nslots = out_ref.shape[0]
        out_ref[...] = slots_ref[...] * (acc_ref[0:nslots, :] * inv_n)


def kernel(input_stream, ssm_slots, msm_slots, router_w, router_b,
           compress_w, compress_b):
    del compress_w, compress_b
    n, d = input_stream.shape
    nslots = router_w.shape[0]

    chunk = 8192
    grid = n // chunk

    w_pad = (jnp.zeros((SUB, d), jnp.float32).at[:nslots, :]
             .set(router_w).astype(jnp.bfloat16))
    b_pad = jnp.full((SUB, 1), NEG, jnp.float32).at[:nslots, 0].set(router_b)
    slots = jnp.concatenate([ssm_slots, msm_slots], axis=0)

    out = pl.pallas_call(
        functools.partial(_router_kernel, grid=grid, inv_n=1.0 / n),
        grid=(grid,),
        in_specs=[
            pl.BlockSpec((chunk, d), lambda i: (i, 0)),
            pl.BlockSpec((SUB, d), lambda i: (0, 0)),
            pl.BlockSpec((SUB, 1), lambda i: (0, 0)),
            pl.BlockSpec((nslots, d), lambda i: (0, 0)),
        ],
        out_specs=pl.BlockSpec((nslots, d), lambda i: (0, 0)),
        out_shape=jax.ShapeDtypeStruct((nslots, d), jnp.float32),
        scratch_shapes=[pltpu.VMEM((SUB, 1), jnp.float32)],
    )(input_stream, w_pad, b_pad, slots)
    return out
